# Initial kernel scaffold; baseline (speedup 1.0000x reference)
import jax
import jax.numpy as jnp
from jax.experimental import pallas as pl


def kernel(ref1, pos1, neg1, ref2, pos2, neg2):
    raise NotImplementedError("write your pallas kernel here")


# speedup vs baseline: 2.8549x; 2.8549x over previous
import functools

import jax
import jax.numpy as jnp
from jax import lax
from jax.experimental import pallas as pl
from jax.experimental.pallas import tpu as pltpu

_LOG2E = 1.4426950408889634
_LN2 = 0.6931471805599453
_INV_TEMPS = (1.0, 2.0)
_TM = 1024


def _body(ref1_ref, pos1_ref, neg1_ref, ref2_ref, pos2_ref, neg2_ref,
          out_ref, r1s, r2s, m1, l1, m2, l2, *, n, tm, num_mt):
    mi = pl.program_id(0)

    @pl.when(mi == 0)
    def _init():
        r1s[...] = (ref1_ref[...] * jnp.float32(_INV_TEMPS[0] * _LOG2E)
                    ).astype(jnp.bfloat16)
        r2s[...] = (ref2_ref[...] * jnp.float32(_INV_TEMPS[1] * _LOG2E)
                    ).astype(jnp.bfloat16)
        for m_scr, l_scr in ((m1, l1), (m2, l2)):
            m_scr[...] = jnp.full(m_scr.shape, -jnp.inf, dtype=jnp.bfloat16)
            l_scr[...] = jnp.zeros(l_scr.shape, dtype=jnp.float32)

    crits = ((r1s, neg1_ref, m1, l1), (r2s, neg2_ref, m2, l2))
    staged = []
    for refs_s, neg_ref, m_scr, l_scr in crits:
        scores = lax.dot_general(
            refs_s[...], neg_ref[...].astype(jnp.bfloat16),
            dimension_numbers=(((1,), (1,)), ((), ())),
            preferred_element_type=jnp.float32,
        ).astype(jnp.bfloat16)
        chunks = [scores[:, g * 128:(g + 1) * 128] for g in range(tm // 128)]
        tile_m = chunks[0]
        for c in chunks[1:]:
            tile_m = jnp.maximum(tile_m, c)
        m_prev = m_scr[...]
        m_new = jnp.maximum(m_prev, tile_m)
        staged.append((chunks, m_prev, m_new, m_scr, l_scr))
    for chunks, m_prev, m_new, m_scr, l_scr in staged:
        part = jnp.exp2(chunks[0] - m_new)
        for c in chunks[1:]:
            part = part + jnp.exp2(c - m_new)
        alpha = jnp.exp2((m_prev - m_new).astype(jnp.float32))
        l_scr[...] = alpha * l_scr[...] + part.astype(jnp.float32)
        m_scr[...] = m_new

    @pl.when(mi == num_mt - 1)
    def _finalize():
        finals = (
            (ref1_ref, pos1_ref, m1, l1, _INV_TEMPS[0] * _LOG2E, 0),
            (ref2_ref, pos2_ref, m2, l2, _INV_TEMPS[1] * _LOG2E, 1),
        )
        inv_n = jnp.float32(1.0 / n)
        for ref_ref, pos_ref, m_scr, l_scr, scale, k in finals:
            m_f32 = m_scr[...].astype(jnp.float32)
            m_row = jnp.max(m_f32, axis=-1, keepdims=True)
            l_row = jnp.sum(l_scr[...] * jnp.exp2(m_f32 - m_row),
                            axis=-1, keepdims=True)
            pos_dist = jnp.sum(ref_ref[...] * jnp.float32(scale) * pos_ref[...],
                               axis=-1, keepdims=True)
            align = jnp.sum(m_row - pos_dist) * jnp.float32(_LN2) * inv_n
            uniform = jnp.sum(jnp.log(l_row)) * inv_n
            out_ref[k, 0] = align + uniform
            out_ref[k, 1] = align
            out_ref[k, 2] = uniform


def kernel(ref1, pos1, neg1, ref2, pos2, neg2):
    n, d = ref1.shape
    m = neg1.shape[0]
    tm = _TM if m % _TM == 0 else m
    num_mt = m // tm

    row_spec = pl.BlockSpec((n, d), lambda mi: (0, 0))
    neg_spec = pl.BlockSpec((tm, d), lambda mi: (mi, 0))
    body = functools.partial(_body, n=n, tm=tm, num_mt=num_mt)
    return pl.pallas_call(
        body,
        grid=(num_mt,),
        in_specs=[row_spec, row_spec, neg_spec, row_spec, row_spec, neg_spec],
        out_specs=pl.BlockSpec(memory_space=pltpu.MemorySpace.SMEM),
        out_shape=jax.ShapeDtypeStruct((2, 3), jnp.float32),
        scratch_shapes=[
            pltpu.VMEM((n, d), jnp.bfloat16),
            pltpu.VMEM((n, d), jnp.bfloat16),
            pltpu.VMEM((n, 128), jnp.bfloat16),
            pltpu.VMEM((n, 128), jnp.float32),
            pltpu.VMEM((n, 128), jnp.bfloat16),
            pltpu.VMEM((n, 128), jnp.float32),
        ],
        compiler_params=pltpu.CompilerParams(
            dimension_semantics=("arbitrary",)),
    )(ref1, pos1, neg1, ref2, pos2, neg2)
